# Initial kernel scaffold; baseline (speedup 1.0000x reference)
import jax
import jax.numpy as jnp
from jax.experimental import pallas as pl


def kernel(kg_edge_index, kg_edge_type, ui_edge_index, all_embed, all_embed_cf, dropout):
    raise NotImplementedError("write your pallas kernel here")


# speedup vs baseline: 4.8129x; 4.8129x over previous
import functools

import jax
import jax.numpy as jnp
from jax import lax
from jax.experimental import pallas as pl
from jax.experimental.pallas import tpu as pltpu
from jax.experimental.pallas import tpu_sc as plsc

_DIM = 128
_N_USERS = 20000
_N_ITEMS = 8000
_N_ENT = 10000
_N_REL = 4
_K_ATT = 0.5
_E_KG = 320000
_E_UI = 500000

_NPE = 10240
_SENT_E = 10200
_NPU = 8192
_SENT_U = 8100
_NBINS = _N_REL * _NPE + _NPU
_SENT_B = _N_REL * _NPE + _SENT_U

_KG_SUB = 80
_EPKG = 32 * _KG_SUB * 128
_UI_SUB = 248
_EPUI = 16 * _UI_SUB * 128
_HROWS = 208
_EHIST = 32 * _HROWS * 128

_mesh = plsc.VectorSubcoreMesh(core_axis_name="c", subcore_axis_name="s")
_sc_params = pltpu.CompilerParams(needs_layout_passes=False)


@functools.partial(
    pl.kernel,
    out_type=jax.ShapeDtypeStruct((2 * _NBINS,), jnp.float32),
    mesh=_mesh,
    compiler_params=_sc_params,
    scratch_types=[
        pltpu.VMEM_SHARED((_NBINS,), jnp.float32),
        pltpu.VMEM((128,), jnp.int32),
        pltpu.VMEM((128,), jnp.float32),
        pltpu.VMEM((_NBINS // 16,), jnp.float32),
    ],
)
def _hist(hidx_h, out_h, cnt_sh, hidx_v, ones_v, zv):
    c = lax.axis_index("c")
    s = lax.axis_index("s")
    wid = s * 2 + c
    one = jnp.ones((16,), jnp.float32)
    for i in range(8):
        ones_v[pl.ds(i * 16, 16)] = one
    z = jnp.zeros((16,), jnp.float32)
    def zb(i, _):
        zv[pl.ds(i * 16, 16)] = z
        return 0
    lax.fori_loop(0, _NBINS // 16 // 16, zb, 0)
    pltpu.sync_copy(zv, cnt_sh.at[pl.ds(s * (_NBINS // 16), _NBINS // 16)])
    plsc.subcore_barrier()
    base = wid * _HROWS * 128
    def hb(r, _):
        pltpu.sync_copy(hidx_h.at[pl.ds(base + r * 128, 128)], hidx_v)
        pltpu.sync_copy(ones_v, cnt_sh.at[hidx_v], add=True)
        return 0
    lax.fori_loop(0, _HROWS, hb, 0)
    plsc.subcore_barrier()
    sl = _NBINS // 16
    pltpu.sync_copy(cnt_sh.at[pl.ds(s * sl, sl)],
                    out_h.at[pl.ds(c * _NBINS + s * sl, sl)])


@functools.partial(
    pl.kernel,
    out_type=jax.ShapeDtypeStruct((_EPKG,), jnp.float32),
    mesh=_mesh,
    compiler_params=_sc_params,
    scratch_types=[
        pltpu.VMEM((_N_REL * _NPE,), jnp.float32),
        pltpu.VMEM((128,), jnp.int32),
        pltpu.VMEM((128,), jnp.float32),
    ],
)
def _wts(td_h, recip_h, whop_h, rtab, tdv, wv):
    c = lax.axis_index("c")
    s = lax.axis_index("s")
    wid = s * 2 + c
    pltpu.sync_copy(recip_h, rtab)
    base = wid * _KG_SUB * 128
    def chunk(cc, _):
        eoff = base + cc * 128
        pltpu.sync_copy(td_h.at[pl.ds(eoff, 128)], tdv)
        def wb(i, _):
            idx16 = tdv[pl.ds(i * 16, 16)]
            wv[pl.ds(i * 16, 16)] = plsc.load_gather(rtab, [idx16])
            return 0
        lax.fori_loop(0, 8, wb, 0)
        pltpu.sync_copy(wv, whop_h.at[pl.ds(eoff, 128)])
        return 0
    lax.fori_loop(0, _KG_SUB, chunk, 0)


@functools.partial(
    pl.kernel,
    out_type=jax.ShapeDtypeStruct((2 * _NPE, _DIM), jnp.float32),
    mesh=_mesh,
    compiler_params=_sc_params,
    scratch_types=[
        pltpu.VMEM_SHARED((_NPE, _DIM), jnp.float32),
        pltpu.VMEM((128, _DIM), jnp.float32),
        pltpu.VMEM((128,), jnp.int32),
        pltpu.VMEM((128,), jnp.int32),
        pltpu.VMEM((128,), jnp.int32),
        pltpu.VMEM((128,), jnp.float32),
        pltpu.SemaphoreType.DMA,
    ],
)
def _passb(src_h, dst_h, ty_h, whop_h, emb_h, part_h,
           acc_sh, rows, sidx, didx, tyv, wv, sem):
    c = lax.axis_index("c")
    s = lax.axis_index("s")
    wid = s * 2 + c
    z = jnp.zeros((16,), jnp.float32)
    def zb(i, _):
        for k in range(8):
            rows[i, pl.ds(k * 16, 16)] = z
        return 0
    lax.fori_loop(0, 128, zb, 0)
    for i in range(5):
        pltpu.sync_copy(rows, acc_sh.at[pl.ds(s * 640 + i * 128, 128)])
    plsc.subcore_barrier()

    base = wid * _KG_SUB * 128
    def chunk(cc, _):
        eoff = base + cc * 128
        pltpu.sync_copy(src_h.at[pl.ds(eoff, 128)], sidx)
        pltpu.sync_copy(dst_h.at[pl.ds(eoff, 128)], didx)
        pltpu.sync_copy(ty_h.at[pl.ds(eoff, 128)], tyv)
        pltpu.sync_copy(whop_h.at[pl.ds(eoff, 128)], wv)
        pltpu.async_copy(emb_h.at[sidx], rows, sem).wait()
        def sb(g, _):
            w16 = wv[pl.ds(g * 16, 16)]
            t16 = tyv[pl.ds(g * 16, 16)]
            for j in range(16):
                w = w16[j]
                t = t16[j]
                e = g * 16 + j
                for k in range(8):
                    wk = jnp.where(t == (k // 2), w, jnp.float32(0.0))
                    rows[e, pl.ds(k * 16, 16)] = rows[e, pl.ds(k * 16, 16)] * wk
            return 0
        lax.fori_loop(0, 8, sb, 0)
        pltpu.sync_copy(rows, acc_sh.at[didx], add=True)
        return 0
    lax.fori_loop(0, _KG_SUB, chunk, 0)
    plsc.subcore_barrier()
    pltpu.sync_copy(acc_sh.at[pl.ds(s * 640, 640)],
                    part_h.at[pl.ds(c * _NPE + s * 640, 640)])


@functools.partial(
    pl.kernel,
    out_type=jax.ShapeDtypeStruct((2 * _NPE, _DIM), jnp.float32),
    mesh=_mesh,
    compiler_params=_sc_params,
    scratch_types=[
        pltpu.VMEM_SHARED((_NPE, _DIM), jnp.float32),
        pltpu.VMEM((128, _DIM), jnp.float32),
        pltpu.VMEM((128,), jnp.int32),
        pltpu.VMEM((128,), jnp.int32),
        pltpu.VMEM((128,), jnp.float32),
        pltpu.SemaphoreType.DMA,
    ],
)
def _hop(src_h, dst_h, whop_h, tab_h, part_h,
         acc_sh, rows, sidx, didx, wv, sem):
    c = lax.axis_index("c")
    s = lax.axis_index("s")
    wid = s * 2 + c
    z = jnp.zeros((16,), jnp.float32)
    def zb(i, _):
        for k in range(8):
            rows[i, pl.ds(k * 16, 16)] = z
        return 0
    lax.fori_loop(0, 128, zb, 0)
    for i in range(5):
        pltpu.sync_copy(rows, acc_sh.at[pl.ds(s * 640 + i * 128, 128)])
    plsc.subcore_barrier()

    base = wid * _KG_SUB * 128
    def chunk(cc, _):
        eoff = base + cc * 128
        pltpu.sync_copy(src_h.at[pl.ds(eoff, 128)], sidx)
        pltpu.sync_copy(dst_h.at[pl.ds(eoff, 128)], didx)
        pltpu.sync_copy(whop_h.at[pl.ds(eoff, 128)], wv)
        pltpu.async_copy(tab_h.at[sidx], rows, sem).wait()
        def sb(g, _):
            w16 = wv[pl.ds(g * 16, 16)]
            for j in range(16):
                w = w16[j]
                e = g * 16 + j
                for k in range(8):
                    rows[e, pl.ds(k * 16, 16)] = rows[e, pl.ds(k * 16, 16)] * w
            return 0
        lax.fori_loop(0, 8, sb, 0)
        pltpu.sync_copy(rows, acc_sh.at[didx], add=True)
        return 0
    lax.fori_loop(0, _KG_SUB, chunk, 0)
    plsc.subcore_barrier()
    pltpu.sync_copy(acc_sh.at[pl.ds(s * 640, 640)],
                    part_h.at[pl.ds(c * _NPE + s * 640, 640)])


@functools.partial(
    pl.kernel,
    out_type=jax.ShapeDtypeStruct((2 * _NPU, _DIM), jnp.float32),
    mesh=_mesh,
    compiler_params=_sc_params,
    scratch_types=[
        pltpu.VMEM_SHARED((_NPU, _DIM), jnp.float32),
        pltpu.VMEM((128, _DIM), jnp.float32),
        pltpu.VMEM((128,), jnp.int32),
        pltpu.VMEM((128,), jnp.int32),
        pltpu.SemaphoreType.DMA,
    ],
)
def _ui(isrc_h, udst_h, res_h, cf_h, usum_h,
        acc_sh, rows, sidx, didx, sem):
    c = lax.axis_index("c")
    s = lax.axis_index("s")
    z = jnp.zeros((16,), jnp.float32)
    def zb(i, _):
        for k in range(8):
            rows[i, pl.ds(k * 16, 16)] = z
        return 0
    lax.fori_loop(0, 128, zb, 0)
    for i in range(4):
        pltpu.sync_copy(rows, acc_sh.at[pl.ds(s * 512 + i * 128, 128)])
    plsc.subcore_barrier()

    off = c * 20000
    base = s * _UI_SUB * 128
    def chunk(cc, _):
        eoff = base + cc * 128
        pltpu.sync_copy(isrc_h.at[pl.ds(eoff, 128)], sidx)
        pltpu.sync_copy(udst_h.at[pl.ds(eoff, 128)], didx)
        for i in range(8):
            sidx[pl.ds(i * 16, 16)] = sidx[pl.ds(i * 16, 16)] + off
        @pl.when(c == 0)
        def _():
            pltpu.async_copy(res_h.at[sidx], rows, sem).wait()
        @pl.when(c == 1)
        def _():
            pltpu.async_copy(cf_h.at[sidx], rows, sem).wait()
        pltpu.sync_copy(rows, acc_sh.at[didx], add=True)
        return 0
    lax.fori_loop(0, _UI_SUB, chunk, 0)
    plsc.subcore_barrier()
    pltpu.sync_copy(acc_sh.at[pl.ds(s * 512, 512)],
                    usum_h.at[pl.ds(c * _NPU + s * 512, 512)])


def _recip_body(p_ref, o_ref):
    cnt = p_ref[0] + p_ref[1]
    o_ref[...] = 1.0 / jnp.maximum(cnt, 1.0)


def _recip(parts):
    p = parts.reshape(2, _NBINS // 128, 128)
    out = pl.pallas_call(
        _recip_body,
        out_shape=jax.ShapeDtypeStruct((_NBINS // 128, 128), jnp.float32),
    )(p)
    return out.reshape(_NBINS)


def _comb_body(p0_ref, p1_ref, r_ref, t_ref, ro_ref):
    t = p0_ref[...] + p1_ref[...]
    t_ref[...] = t
    ro_ref[...] = r_ref[...] + t


def _combine(part, res_in):
    blk = 2048
    grid = _NPE // blk
    spec = pl.BlockSpec((blk, _DIM), lambda i: (i, 0))
    return pl.pallas_call(
        _comb_body,
        grid=(grid,),
        in_specs=[spec, spec, spec],
        out_specs=[spec, spec],
        out_shape=[
            jax.ShapeDtypeStruct((_NPE, _DIM), jnp.float32),
            jax.ShapeDtypeStruct((_NPE, _DIM), jnp.float32),
        ],
    )(part[:_NPE], part[_NPE:], res_in)


def _mean_body(r_ref, o_ref):
    rid = lax.broadcasted_iota(jnp.int32, (_NPE, _DIM), 0)
    m = jnp.sum(jnp.where(rid < _N_ITEMS, r_ref[...], 0.0), axis=0,
                keepdims=True) * (1.0 / _N_ITEMS)
    o_ref[...] = jnp.broadcast_to(m, (8, _DIM))


def _item_mean(res):
    return pl.pallas_call(
        _mean_body,
        out_shape=jax.ShapeDtypeStruct((8, _DIM), jnp.float32),
    )(res)


def _users_body(sf_ref, sf0_ref, rc_ref, ucf_ref, mi_ref, o_ref):
    rc = rc_ref[...]
    uf = sf_ref[...] * rc
    uf0 = sf0_ref[...] * rc
    ucf = ucf_ref[...]
    mi = mi_ref[0:1, :]
    cols = []
    for i in range(_N_REL):
        a, b = 32 * i, 32 * (i + 1)
        ua = jnp.sum(uf[:, a:b] * ucf[:, a:b], axis=1, keepdims=True)
        ua = jnp.maximum(ua, 0.0) + 1e-10
        ma = jnp.sum(mi[:, a:b] * ucf[:, a:b], axis=1, keepdims=True)
        ma = jnp.maximum(ma, 0.0) + 1e-08
        att = _K_ATT * jnp.maximum(ua / ma - 1.0, 0.0) + 0.01
        sc = jnp.tanh(att)
        cols.append(jnp.broadcast_to(sc, (sc.shape[0], 32)))
    score = jnp.concatenate(cols, axis=1)
    o_ref[...] = score * uf + uf0 + ucf


def _users(sf, sf0, rc, ucf, mi):
    blk = 4000
    grid = _N_USERS // blk
    spec = pl.BlockSpec((blk, _DIM), lambda i: (i, 0))
    spec1 = pl.BlockSpec((blk, 1), lambda i: (i, 0))
    specm = pl.BlockSpec((8, _DIM), lambda i: (0, 0))
    return pl.pallas_call(
        _users_body,
        grid=(grid,),
        in_specs=[spec, spec, spec1, spec, specm],
        out_specs=spec,
        out_shape=jax.ShapeDtypeStruct((_N_USERS, _DIM), jnp.float32),
    )(sf, sf0, rc, ucf, mi)


def _ent_body(r_ref, c_ref, o_ref):
    o_ref[...] = r_ref[...] + c_ref[...]


def _entity(res8k, cfe):
    blk = 2000
    grid = _N_ITEMS // blk
    spec = pl.BlockSpec((blk, _DIM), lambda i: (i, 0))
    return pl.pallas_call(
        _ent_body,
        grid=(grid,),
        in_specs=[spec, spec],
        out_specs=spec,
        out_shape=jax.ShapeDtypeStruct((_N_ITEMS, _DIM), jnp.float32),
    )(res8k, cfe)


def kernel(kg_edge_index, kg_edge_type, ui_edge_index, all_embed,
           all_embed_cf, dropout):
    src = kg_edge_index[0]
    dst = kg_edge_index[1]
    typ = kg_edge_type
    pad_kg = _EPKG - _E_KG
    src_p = jnp.concatenate([src, jnp.zeros((pad_kg,), jnp.int32)])
    dst_p = jnp.concatenate([dst, jnp.full((pad_kg,), _SENT_E, jnp.int32)])
    typ_p = jnp.concatenate([typ, jnp.zeros((pad_kg,), jnp.int32)])
    td_p = typ_p * _NPE + dst_p

    isrc = ui_edge_index[0]
    udst = ui_edge_index[1]
    pad_ui = _EPUI - _E_UI
    isrc_p = jnp.concatenate([isrc, jnp.zeros((pad_ui,), jnp.int32)])
    udst_p = jnp.concatenate([udst, jnp.full((pad_ui,), _SENT_U, jnp.int32)])

    pad_h = _EHIST - (_EPKG + _EPUI)
    hidx = jnp.concatenate(
        [td_p, udst_p + _N_REL * _NPE,
         jnp.full((pad_h,), _SENT_B, jnp.int32)])

    cnt_parts = _hist(hidx)
    recip = _recip(cnt_parts.reshape(2, _NBINS))
    recip_kg = recip[: _N_REL * _NPE]

    whop = _wts(td_p, recip_kg)
    part0 = _passb(src_p, dst_p, typ_p, whop, all_embed)
    e0, res = _combine(part0, jnp.zeros((_NPE, _DIM), jnp.float32))
    part1 = _hop(src_p, dst_p, whop, e0)
    c1, res = _combine(part1, res)
    part2 = _hop(src_p, dst_p, whop, c1)
    _, res = _combine(part2, res)

    usum = _ui(isrc_p, udst_p, res, all_embed_cf)

    sf = jnp.pad(usum[:_N_ITEMS], ((0, _N_USERS - _N_ITEMS), (0, 0)))
    sf0 = jnp.pad(usum[_NPU:_NPU + _N_ITEMS],
                  ((0, _N_USERS - _N_ITEMS), (0, 0)))
    rcu = jnp.pad(recip[_N_REL * _NPE:_N_REL * _NPE + _N_ITEMS],
                  (0, _N_USERS - _N_ITEMS)).reshape(_N_USERS, 1)

    mi = _item_mean(res)
    ucf = all_embed_cf[:_N_USERS]
    user_out = _users(sf, sf0, rcu, ucf, mi)
    ent_out = _entity(res[:_N_ITEMS], all_embed_cf[_N_USERS:_N_USERS + _N_ITEMS])
    return jnp.concatenate([user_out, ent_out], axis=0)


# speedup vs baseline: 5.7088x; 1.1861x over previous
import functools

import jax
import jax.numpy as jnp
from jax import lax
from jax.experimental import pallas as pl
from jax.experimental.pallas import tpu as pltpu
from jax.experimental.pallas import tpu_sc as plsc

_DIM = 128
_N_USERS = 20000
_N_ITEMS = 8000
_N_ENT = 10000
_N_REL = 4
_K_ATT = 0.5
_E_KG = 320000
_E_UI = 500000

_NPE = 10240
_SENT_E = 10200
_NPU = 8192
_SENT_U = 8100
_NBINS = _N_REL * _NPE + _NPU
_SENT_B = _N_REL * _NPE + _SENT_U

_KG_SUB = 80
_EPKG = 32 * _KG_SUB * 128
_UI_SUB = 248
_EPUI = 16 * _UI_SUB * 128
_HROWS = 208
_EHIST = 32 * _HROWS * 128

_mesh = plsc.VectorSubcoreMesh(core_axis_name="c", subcore_axis_name="s")
_sc_params = pltpu.CompilerParams(needs_layout_passes=False)


@functools.partial(
    pl.kernel,
    out_type=jax.ShapeDtypeStruct((2 * _NBINS,), jnp.float32),
    mesh=_mesh,
    compiler_params=_sc_params,
    scratch_types=[
        pltpu.VMEM_SHARED((_NBINS,), jnp.float32),
        pltpu.VMEM((128,), jnp.int32),
        pltpu.VMEM((128,), jnp.float32),
        pltpu.VMEM((_NBINS // 16,), jnp.float32),
    ],
)
def _hist(hidx_h, out_h, cnt_sh, hidx_v, ones_v, zv):
    c = lax.axis_index("c")
    s = lax.axis_index("s")
    wid = s * 2 + c
    one = jnp.ones((16,), jnp.float32)
    for i in range(8):
        ones_v[pl.ds(i * 16, 16)] = one
    z = jnp.zeros((16,), jnp.float32)
    def zb(i, _):
        zv[pl.ds(i * 16, 16)] = z
        return 0
    lax.fori_loop(0, _NBINS // 16 // 16, zb, 0)
    pltpu.sync_copy(zv, cnt_sh.at[pl.ds(s * (_NBINS // 16), _NBINS // 16)])
    plsc.subcore_barrier()
    base = wid * _HROWS * 128
    def hb(r, _):
        pltpu.sync_copy(hidx_h.at[pl.ds(base + r * 128, 128)], hidx_v)
        pltpu.sync_copy(ones_v, cnt_sh.at[hidx_v], add=True)
        return 0
    lax.fori_loop(0, _HROWS, hb, 0)
    plsc.subcore_barrier()
    sl = _NBINS // 16
    pltpu.sync_copy(cnt_sh.at[pl.ds(s * sl, sl)],
                    out_h.at[pl.ds(c * _NBINS + s * sl, sl)])


@functools.partial(
    pl.kernel,
    out_type=jax.ShapeDtypeStruct((_EPKG,), jnp.float32),
    mesh=_mesh,
    compiler_params=_sc_params,
    scratch_types=[
        pltpu.VMEM((_N_REL * _NPE,), jnp.float32),
        pltpu.VMEM((_KG_SUB * 128,), jnp.int32),
        pltpu.VMEM((_KG_SUB * 128,), jnp.float32),
    ],
)
def _wts(td_h, recip_h, whop_h, rtab, tdv, wv):
    c = lax.axis_index("c")
    s = lax.axis_index("s")
    wid = s * 2 + c
    n = _KG_SUB * 128
    pltpu.sync_copy(recip_h, rtab)
    pltpu.sync_copy(td_h.at[pl.ds(wid * n, n)], tdv)
    def wb(i, _):
        idx16 = tdv[pl.ds(i * 16, 16)]
        wv[pl.ds(i * 16, 16)] = plsc.load_gather(rtab, [idx16])
        return 0
    lax.fori_loop(0, n // 16, wb, 0)
    pltpu.sync_copy(wv, whop_h.at[pl.ds(wid * n, n)])


@functools.partial(
    pl.kernel,
    out_type=jax.ShapeDtypeStruct((2 * _NPE, _DIM), jnp.float32),
    mesh=_mesh,
    compiler_params=_sc_params,
    scratch_types=[
        pltpu.VMEM_SHARED((_NPE, _DIM), jnp.float32),
        pltpu.VMEM((_KG_SUB * 128,), jnp.int32),
        pltpu.VMEM((_KG_SUB * 128,), jnp.float32),
        pltpu.VMEM((128, _DIM), jnp.float32),
        pltpu.VMEM((128,), jnp.int32),
        pltpu.VMEM((128,), jnp.int32),
        pltpu.SemaphoreType.DMA,
    ],
)
def _passb(pck_h, whop_h, emb_h, part_h,
           acc_sh, pck, wvb, rows, sidx, didx, sem):
    c = lax.axis_index("c")
    s = lax.axis_index("s")
    wid = s * 2 + c
    n = _KG_SUB * 128
    pltpu.sync_copy(pck_h.at[pl.ds(wid * n, n)], pck)
    pltpu.sync_copy(whop_h.at[pl.ds(wid * n, n)], wvb)
    z = jnp.zeros((16,), jnp.float32)
    def zb(i, _):
        for k in range(8):
            rows[i, pl.ds(k * 16, 16)] = z
        return 0
    lax.fori_loop(0, 128, zb, 0)
    for i in range(5):
        pltpu.sync_copy(rows, acc_sh.at[pl.ds(s * 640 + i * 128, 128)])
    plsc.subcore_barrier()

    def chunk(cc, _):
        def ub(i, _):
            p16 = pck[pl.ds(cc * 128 + i * 16, 16)]
            sidx[pl.ds(i * 16, 16)] = p16 & 16383
            didx[pl.ds(i * 16, 16)] = (p16 >> 14) & 16383
            return 0
        lax.fori_loop(0, 8, ub, 0)
        pltpu.async_copy(emb_h.at[sidx], rows, sem).wait()
        def sb(g, _):
            w16 = wvb[pl.ds(cc * 128 + g * 16, 16)]
            t16 = pck[pl.ds(cc * 128 + g * 16, 16)] >> 28
            for j in range(16):
                w = w16[j]
                t = t16[j]
                e = g * 16 + j
                for k in range(8):
                    wk = jnp.where(t == (k // 2), w, jnp.float32(0.0))
                    rows[e, pl.ds(k * 16, 16)] = rows[e, pl.ds(k * 16, 16)] * wk
            return 0
        lax.fori_loop(0, 8, sb, 0)
        pltpu.sync_copy(rows, acc_sh.at[didx], add=True)
        return 0
    lax.fori_loop(0, _KG_SUB, chunk, 0)
    plsc.subcore_barrier()
    pltpu.sync_copy(acc_sh.at[pl.ds(s * 640, 640)],
                    part_h.at[pl.ds(c * _NPE + s * 640, 640)])


@functools.partial(
    pl.kernel,
    out_type=jax.ShapeDtypeStruct((2 * _NPE, _DIM), jnp.float32),
    mesh=_mesh,
    compiler_params=_sc_params,
    scratch_types=[
        pltpu.VMEM_SHARED((_NPE, _DIM), jnp.float32),
        pltpu.VMEM((_KG_SUB * 128,), jnp.int32),
        pltpu.VMEM((_KG_SUB * 128,), jnp.float32),
        pltpu.VMEM((128, _DIM), jnp.float32),
        pltpu.VMEM((128,), jnp.int32),
        pltpu.VMEM((128,), jnp.int32),
        pltpu.SemaphoreType.DMA,
    ],
)
def _hop(pck_h, whop_h, tab_h, part_h,
         acc_sh, pck, wvb, rows, sidx, didx, sem):
    c = lax.axis_index("c")
    s = lax.axis_index("s")
    wid = s * 2 + c
    n = _KG_SUB * 128
    pltpu.sync_copy(pck_h.at[pl.ds(wid * n, n)], pck)
    pltpu.sync_copy(whop_h.at[pl.ds(wid * n, n)], wvb)
    z = jnp.zeros((16,), jnp.float32)
    def zb(i, _):
        for k in range(8):
            rows[i, pl.ds(k * 16, 16)] = z
        return 0
    lax.fori_loop(0, 128, zb, 0)
    for i in range(5):
        pltpu.sync_copy(rows, acc_sh.at[pl.ds(s * 640 + i * 128, 128)])
    plsc.subcore_barrier()

    def chunk(cc, _):
        def ub(i, _):
            p16 = pck[pl.ds(cc * 128 + i * 16, 16)]
            sidx[pl.ds(i * 16, 16)] = p16 & 16383
            didx[pl.ds(i * 16, 16)] = (p16 >> 14) & 16383
            return 0
        lax.fori_loop(0, 8, ub, 0)
        pltpu.async_copy(tab_h.at[sidx], rows, sem).wait()
        def sb(g, _):
            w16 = wvb[pl.ds(cc * 128 + g * 16, 16)]
            for j in range(16):
                w = w16[j]
                e = g * 16 + j
                for k in range(8):
                    rows[e, pl.ds(k * 16, 16)] = rows[e, pl.ds(k * 16, 16)] * w
            return 0
        lax.fori_loop(0, 8, sb, 0)
        pltpu.sync_copy(rows, acc_sh.at[didx], add=True)
        return 0
    lax.fori_loop(0, _KG_SUB, chunk, 0)
    plsc.subcore_barrier()
    pltpu.sync_copy(acc_sh.at[pl.ds(s * 640, 640)],
                    part_h.at[pl.ds(c * _NPE + s * 640, 640)])


@functools.partial(
    pl.kernel,
    out_type=jax.ShapeDtypeStruct((2 * _NPU, _DIM), jnp.float32),
    mesh=_mesh,
    compiler_params=_sc_params,
    scratch_types=[
        pltpu.VMEM_SHARED((_NPU, _DIM), jnp.float32),
        pltpu.VMEM((_UI_SUB * 128,), jnp.int32),
        pltpu.VMEM((128, _DIM), jnp.float32),
        pltpu.VMEM((128,), jnp.int32),
        pltpu.VMEM((128,), jnp.int32),
        pltpu.SemaphoreType.DMA,
    ],
)
def _ui(pck_h, res_h, cf_h, usum_h,
        acc_sh, pck, rows, sidx, didx, sem):
    c = lax.axis_index("c")
    s = lax.axis_index("s")
    n = _UI_SUB * 128
    pltpu.sync_copy(pck_h.at[pl.ds(s * n, n)], pck)
    z = jnp.zeros((16,), jnp.float32)
    def zb(i, _):
        for k in range(8):
            rows[i, pl.ds(k * 16, 16)] = z
        return 0
    lax.fori_loop(0, 128, zb, 0)
    for i in range(4):
        pltpu.sync_copy(rows, acc_sh.at[pl.ds(s * 512 + i * 128, 128)])
    plsc.subcore_barrier()

    off = c * 20000
    def chunk(cc, _):
        def ub(i, _):
            p16 = pck[pl.ds(cc * 128 + i * 16, 16)]
            sidx[pl.ds(i * 16, 16)] = (p16 & 8191) + off
            didx[pl.ds(i * 16, 16)] = p16 >> 13
            return 0
        lax.fori_loop(0, 8, ub, 0)
        @pl.when(c == 0)
        def _():
            pltpu.async_copy(res_h.at[sidx], rows, sem).wait()
        @pl.when(c == 1)
        def _():
            pltpu.async_copy(cf_h.at[sidx], rows, sem).wait()
        pltpu.sync_copy(rows, acc_sh.at[didx], add=True)
        return 0
    lax.fori_loop(0, _UI_SUB, chunk, 0)
    plsc.subcore_barrier()
    pltpu.sync_copy(acc_sh.at[pl.ds(s * 512, 512)],
                    usum_h.at[pl.ds(c * _NPU + s * 512, 512)])


def _recip_body(p_ref, o_ref):
    cnt = p_ref[0] + p_ref[1]
    o_ref[...] = 1.0 / jnp.maximum(cnt, 1.0)


def _recip(parts):
    p = parts.reshape(2, _NBINS // 128, 128)
    out = pl.pallas_call(
        _recip_body,
        out_shape=jax.ShapeDtypeStruct((_NBINS // 128, 128), jnp.float32),
    )(p)
    return out.reshape(_NBINS)


def _comb_body(p0_ref, p1_ref, r_ref, t_ref, ro_ref):
    t = p0_ref[...] + p1_ref[...]
    t_ref[...] = t
    ro_ref[...] = r_ref[...] + t


def _combine(part, res_in):
    blk = 2048
    grid = _NPE // blk
    spec = pl.BlockSpec((blk, _DIM), lambda i: (i, 0))
    return pl.pallas_call(
        _comb_body,
        grid=(grid,),
        in_specs=[spec, spec, spec],
        out_specs=[spec, spec],
        out_shape=[
            jax.ShapeDtypeStruct((_NPE, _DIM), jnp.float32),
            jax.ShapeDtypeStruct((_NPE, _DIM), jnp.float32),
        ],
    )(part[:_NPE], part[_NPE:], res_in)


def _mean_body(r_ref, o_ref):
    rid = lax.broadcasted_iota(jnp.int32, (_NPE, _DIM), 0)
    m = jnp.sum(jnp.where(rid < _N_ITEMS, r_ref[...], 0.0), axis=0,
                keepdims=True) * (1.0 / _N_ITEMS)
    o_ref[...] = jnp.broadcast_to(m, (8, _DIM))


def _item_mean(res):
    return pl.pallas_call(
        _mean_body,
        out_shape=jax.ShapeDtypeStruct((8, _DIM), jnp.float32),
    )(res)


def _users_body(sf_ref, sf0_ref, rc_ref, ucf_ref, mi_ref, o_ref):
    rc = rc_ref[...]
    uf = sf_ref[...] * rc
    uf0 = sf0_ref[...] * rc
    ucf = ucf_ref[...]
    mi = mi_ref[0:1, :]
    cols = []
    for i in range(_N_REL):
        a, b = 32 * i, 32 * (i + 1)
        ua = jnp.sum(uf[:, a:b] * ucf[:, a:b], axis=1, keepdims=True)
        ua = jnp.maximum(ua, 0.0) + 1e-10
        ma = jnp.sum(mi[:, a:b] * ucf[:, a:b], axis=1, keepdims=True)
        ma = jnp.maximum(ma, 0.0) + 1e-08
        att = _K_ATT * jnp.maximum(ua / ma - 1.0, 0.0) + 0.01
        sc = jnp.tanh(att)
        cols.append(jnp.broadcast_to(sc, (sc.shape[0], 32)))
    score = jnp.concatenate(cols, axis=1)
    o_ref[...] = score * uf + uf0 + ucf


def _users(sf, sf0, rc, ucf, mi):
    blk = 4000
    grid = _N_USERS // blk
    spec = pl.BlockSpec((blk, _DIM), lambda i: (i, 0))
    spec1 = pl.BlockSpec((blk, 1), lambda i: (i, 0))
    specm = pl.BlockSpec((8, _DIM), lambda i: (0, 0))
    return pl.pallas_call(
        _users_body,
        grid=(grid,),
        in_specs=[spec, spec, spec1, spec, specm],
        out_specs=spec,
        out_shape=jax.ShapeDtypeStruct((_N_USERS, _DIM), jnp.float32),
    )(sf, sf0, rc, ucf, mi)


def _ent_body(r_ref, c_ref, o_ref):
    o_ref[...] = r_ref[...] + c_ref[...]


def _entity(res8k, cfe):
    blk = 2000
    grid = _N_ITEMS // blk
    spec = pl.BlockSpec((blk, _DIM), lambda i: (i, 0))
    return pl.pallas_call(
        _ent_body,
        grid=(grid,),
        in_specs=[spec, spec],
        out_specs=spec,
        out_shape=jax.ShapeDtypeStruct((_N_ITEMS, _DIM), jnp.float32),
    )(res8k, cfe)


def kernel(kg_edge_index, kg_edge_type, ui_edge_index, all_embed,
           all_embed_cf, dropout):
    src = kg_edge_index[0]
    dst = kg_edge_index[1]
    typ = kg_edge_type
    pad_kg = _EPKG - _E_KG
    src_p = jnp.concatenate([src, jnp.zeros((pad_kg,), jnp.int32)])
    dst_p = jnp.concatenate([dst, jnp.full((pad_kg,), _SENT_E, jnp.int32)])
    typ_p = jnp.concatenate([typ, jnp.zeros((pad_kg,), jnp.int32)])
    td_p = typ_p * _NPE + dst_p

    isrc = ui_edge_index[0]
    udst = ui_edge_index[1]
    pad_ui = _EPUI - _E_UI
    isrc_p = jnp.concatenate([isrc, jnp.zeros((pad_ui,), jnp.int32)])
    udst_p = jnp.concatenate([udst, jnp.full((pad_ui,), _SENT_U, jnp.int32)])

    pad_h = _EHIST - (_EPKG + _EPUI)
    hidx = jnp.concatenate(
        [td_p, udst_p + _N_REL * _NPE,
         jnp.full((pad_h,), _SENT_B, jnp.int32)])
    pck_kg = src_p + (dst_p << 14) + (typ_p << 28)
    pck_ui = isrc_p + (udst_p << 13)

    cnt_parts = _hist(hidx)
    recip = _recip(cnt_parts.reshape(2, _NBINS))
    recip_kg = recip[: _N_REL * _NPE]

    whop = _wts(td_p, recip_kg)
    part0 = _passb(pck_kg, whop, all_embed)
    e0, res = _combine(part0, jnp.zeros((_NPE, _DIM), jnp.float32))
    part1 = _hop(pck_kg, whop, e0)
    c1, res = _combine(part1, res)
    part2 = _hop(pck_kg, whop, c1)
    _, res = _combine(part2, res)

    usum = _ui(pck_ui, res, all_embed_cf)

    sf = jnp.pad(usum[:_N_ITEMS], ((0, _N_USERS - _N_ITEMS), (0, 0)))
    sf0 = jnp.pad(usum[_NPU:_NPU + _N_ITEMS],
                  ((0, _N_USERS - _N_ITEMS), (0, 0)))
    rcu = jnp.pad(recip[_N_REL * _NPE:_N_REL * _NPE + _N_ITEMS],
                  (0, _N_USERS - _N_ITEMS)).reshape(_N_USERS, 1)

    mi = _item_mean(res)
    ucf = all_embed_cf[:_N_USERS]
    user_out = _users(sf, sf0, rcu, ucf, mi)
    ent_out = _entity(res[:_N_ITEMS], all_embed_cf[_N_USERS:_N_USERS + _N_ITEMS])
    return jnp.concatenate([user_out, ent_out], axis=0)


# speedup vs baseline: 6.5795x; 1.1525x over previous
import functools

import jax
import jax.numpy as jnp
from jax import lax
from jax.experimental import pallas as pl
from jax.experimental.pallas import tpu as pltpu
from jax.experimental.pallas import tpu_sc as plsc

_DIM = 128
_N_USERS = 20000
_N_ITEMS = 8000
_N_ENT = 10000
_N_REL = 4
_K_ATT = 0.5
_E_KG = 320000
_E_UI = 500000

_NPE = 10240
_SENT_E = 10200
_NPU = 8192
_SENT_U = 8100
_NBINS = _N_REL * _NPE + _NPU
_SENT_B = _N_REL * _NPE + _SENT_U

_KG_SUB = 80
_EPKG = 32 * _KG_SUB * 128
_UI_SUB = 248
_EPUI = 16 * _UI_SUB * 128
_HROWS = 208
_EHIST = 32 * _HROWS * 128

_mesh = plsc.VectorSubcoreMesh(core_axis_name="c", subcore_axis_name="s")
_sc_params = pltpu.CompilerParams(needs_layout_passes=False)


@functools.partial(
    pl.kernel,
    out_type=jax.ShapeDtypeStruct((2 * _NBINS,), jnp.float32),
    mesh=_mesh,
    compiler_params=_sc_params,
    scratch_types=[
        pltpu.VMEM_SHARED((_NBINS,), jnp.float32),
        pltpu.VMEM((128,), jnp.int32),
        pltpu.VMEM((128,), jnp.float32),
        pltpu.VMEM((_NBINS // 16,), jnp.float32),
    ],
)
def _hist(hidx_h, out_h, cnt_sh, hidx_v, ones_v, zv):
    c = lax.axis_index("c")
    s = lax.axis_index("s")
    wid = s * 2 + c
    one = jnp.ones((16,), jnp.float32)
    for i in range(8):
        ones_v[pl.ds(i * 16, 16)] = one
    z = jnp.zeros((16,), jnp.float32)
    def zb(i, _):
        zv[pl.ds(i * 16, 16)] = z
        return 0
    lax.fori_loop(0, _NBINS // 16 // 16, zb, 0)
    pltpu.sync_copy(zv, cnt_sh.at[pl.ds(s * (_NBINS // 16), _NBINS // 16)])
    plsc.subcore_barrier()
    base = wid * _HROWS * 128
    def hb(r, _):
        pltpu.sync_copy(hidx_h.at[pl.ds(base + r * 128, 128)], hidx_v)
        pltpu.sync_copy(ones_v, cnt_sh.at[hidx_v], add=True)
        return 0
    lax.fori_loop(0, _HROWS, hb, 0)
    plsc.subcore_barrier()
    sl = _NBINS // 16
    pltpu.sync_copy(cnt_sh.at[pl.ds(s * sl, sl)],
                    out_h.at[pl.ds(c * _NBINS + s * sl, sl)])


@functools.partial(
    pl.kernel,
    out_type=jax.ShapeDtypeStruct((_EPKG,), jnp.float32),
    mesh=_mesh,
    compiler_params=_sc_params,
    scratch_types=[
        pltpu.VMEM((_N_REL * _NPE,), jnp.float32),
        pltpu.VMEM((_KG_SUB * 128,), jnp.int32),
        pltpu.VMEM((_KG_SUB * 128,), jnp.float32),
    ],
)
def _wts(td_h, recip_h, whop_h, rtab, tdv, wv):
    c = lax.axis_index("c")
    s = lax.axis_index("s")
    wid = s * 2 + c
    n = _KG_SUB * 128
    pltpu.sync_copy(recip_h, rtab)
    pltpu.sync_copy(td_h.at[pl.ds(wid * n, n)], tdv)
    def wb(i, _):
        idx16 = tdv[pl.ds(i * 16, 16)]
        wv[pl.ds(i * 16, 16)] = plsc.load_gather(rtab, [idx16])
        return 0
    lax.fori_loop(0, n // 16, wb, 0)
    pltpu.sync_copy(wv, whop_h.at[pl.ds(wid * n, n)])


@functools.partial(
    pl.kernel,
    out_type=jax.ShapeDtypeStruct((2 * _NPE, _DIM), jnp.float32),
    mesh=_mesh,
    compiler_params=_sc_params,
    scratch_types=[
        pltpu.VMEM_SHARED((_NPE, _DIM), jnp.float32),
        pltpu.VMEM((_KG_SUB * 128,), jnp.int32),
        pltpu.VMEM((_KG_SUB * 128,), jnp.float32),
        pltpu.VMEM((64, _DIM), jnp.float32),
        pltpu.VMEM((64, _DIM), jnp.float32),
        pltpu.VMEM((64,), jnp.int32),
        pltpu.VMEM((64,), jnp.int32),
        pltpu.VMEM((64,), jnp.int32),
        pltpu.VMEM((64,), jnp.int32),
        pltpu.SemaphoreType.DMA,
        pltpu.SemaphoreType.DMA,
        pltpu.SemaphoreType.DMA,
        pltpu.SemaphoreType.DMA,
    ],
)
def _passb(pck_h, whop_h, emb_h, part_h,
           acc_sh, pck, wvb, rows0, rows1, sidx0, sidx1, didx0, didx1,
           semg0, semg1, sems0, sems1):
    c = lax.axis_index("c")
    s = lax.axis_index("s")
    wid = s * 2 + c
    n = _KG_SUB * 128
    nch = n // 64
    rows_ = (rows0, rows1)
    sidx_ = (sidx0, sidx1)
    didx_ = (didx0, didx1)
    semg_ = (semg0, semg1)
    sems_ = (sems0, sems1)
    pltpu.sync_copy(pck_h.at[pl.ds(wid * n, n)], pck)
    pltpu.sync_copy(whop_h.at[pl.ds(wid * n, n)], wvb)
    z = jnp.zeros((16,), jnp.float32)
    def zb(i, _):
        for k in range(8):
            rows0[i, pl.ds(k * 16, 16)] = z
        return 0
    lax.fori_loop(0, 64, zb, 0)
    for i in range(10):
        pltpu.sync_copy(rows0, acc_sh.at[pl.ds(s * 640 + i * 64, 64)])
    plsc.subcore_barrier()

    for i in range(4):
        p16 = pck[pl.ds(i * 16, 16)]
        sidx0[pl.ds(i * 16, 16)] = p16 & 16383
        didx0[pl.ds(i * 16, 16)] = (p16 >> 14) & 16383
    pltpu.async_copy(emb_h.at[sidx0], rows0, semg0)

    def grp(g, _):
        for b in range(2):
            b1 = 1 - b
            cc = g * 2 + b
            @pl.when(cc > 0)
            def _():
                pltpu.make_async_copy(rows_[b1], acc_sh.at[didx_[b1]],
                                      sems_[b1]).wait()
            @pl.when(cc + 1 < nch)
            def _():
                def ub(i, _):
                    p16 = pck[pl.ds((cc + 1) * 64 + i * 16, 16)]
                    sidx_[b1][pl.ds(i * 16, 16)] = p16 & 16383
                    didx_[b1][pl.ds(i * 16, 16)] = (p16 >> 14) & 16383
                    return 0
                lax.fori_loop(0, 4, ub, 0)
                pltpu.async_copy(emb_h.at[sidx_[b1]], rows_[b1], semg_[b1])
            pltpu.make_async_copy(emb_h.at[sidx_[b]], rows_[b],
                                  semg_[b]).wait()
            def sb(gg, _):
                w16 = wvb[pl.ds(cc * 64 + gg * 16, 16)]
                t16 = pck[pl.ds(cc * 64 + gg * 16, 16)] >> 28
                for j in range(16):
                    w = w16[j]
                    t = t16[j]
                    e = gg * 16 + j
                    for k in range(8):
                        wk = jnp.where(t == (k // 2), w, jnp.float32(0.0))
                        rows_[b][e, pl.ds(k * 16, 16)] = (
                            rows_[b][e, pl.ds(k * 16, 16)] * wk)
                return 0
            lax.fori_loop(0, 4, sb, 0)
            pltpu.async_copy(rows_[b], acc_sh.at[didx_[b]], sems_[b],
                             add=True)
        return 0
    lax.fori_loop(0, nch // 2, grp, 0)
    pltpu.make_async_copy(rows1, acc_sh.at[didx1], sems1).wait()
    plsc.subcore_barrier()
    pltpu.sync_copy(acc_sh.at[pl.ds(s * 640, 640)],
                    part_h.at[pl.ds(c * _NPE + s * 640, 640)])


@functools.partial(
    pl.kernel,
    out_type=jax.ShapeDtypeStruct((2 * _NPE, _DIM), jnp.float32),
    mesh=_mesh,
    compiler_params=_sc_params,
    scratch_types=[
        pltpu.VMEM_SHARED((_NPE, _DIM), jnp.float32),
        pltpu.VMEM((_KG_SUB * 128,), jnp.int32),
        pltpu.VMEM((_KG_SUB * 128,), jnp.float32),
        pltpu.VMEM((64, _DIM), jnp.float32),
        pltpu.VMEM((64, _DIM), jnp.float32),
        pltpu.VMEM((64,), jnp.int32),
        pltpu.VMEM((64,), jnp.int32),
        pltpu.VMEM((64,), jnp.int32),
        pltpu.VMEM((64,), jnp.int32),
        pltpu.SemaphoreType.DMA,
        pltpu.SemaphoreType.DMA,
        pltpu.SemaphoreType.DMA,
        pltpu.SemaphoreType.DMA,
    ],
)
def _hop(pck_h, whop_h, tab_h, part_h,
         acc_sh, pck, wvb, rows0, rows1, sidx0, sidx1, didx0, didx1,
         semg0, semg1, sems0, sems1):
    c = lax.axis_index("c")
    s = lax.axis_index("s")
    wid = s * 2 + c
    n = _KG_SUB * 128
    nch = n // 64
    rows_ = (rows0, rows1)
    sidx_ = (sidx0, sidx1)
    didx_ = (didx0, didx1)
    semg_ = (semg0, semg1)
    sems_ = (sems0, sems1)
    pltpu.sync_copy(pck_h.at[pl.ds(wid * n, n)], pck)
    pltpu.sync_copy(whop_h.at[pl.ds(wid * n, n)], wvb)
    z = jnp.zeros((16,), jnp.float32)
    def zb(i, _):
        for k in range(8):
            rows0[i, pl.ds(k * 16, 16)] = z
        return 0
    lax.fori_loop(0, 64, zb, 0)
    for i in range(10):
        pltpu.sync_copy(rows0, acc_sh.at[pl.ds(s * 640 + i * 64, 64)])
    plsc.subcore_barrier()

    for i in range(4):
        p16 = pck[pl.ds(i * 16, 16)]
        sidx0[pl.ds(i * 16, 16)] = p16 & 16383
        didx0[pl.ds(i * 16, 16)] = (p16 >> 14) & 16383
    pltpu.async_copy(tab_h.at[sidx0], rows0, semg0)

    def grp(g, _):
        for b in range(2):
            b1 = 1 - b
            cc = g * 2 + b
            @pl.when(cc > 0)
            def _():
                pltpu.make_async_copy(rows_[b1], acc_sh.at[didx_[b1]],
                                      sems_[b1]).wait()
            @pl.when(cc + 1 < nch)
            def _():
                def ub(i, _):
                    p16 = pck[pl.ds((cc + 1) * 64 + i * 16, 16)]
                    sidx_[b1][pl.ds(i * 16, 16)] = p16 & 16383
                    didx_[b1][pl.ds(i * 16, 16)] = (p16 >> 14) & 16383
                    return 0
                lax.fori_loop(0, 4, ub, 0)
                pltpu.async_copy(tab_h.at[sidx_[b1]], rows_[b1], semg_[b1])
            pltpu.make_async_copy(tab_h.at[sidx_[b]], rows_[b],
                                  semg_[b]).wait()
            def sb(gg, _):
                w16 = wvb[pl.ds(cc * 64 + gg * 16, 16)]
                for j in range(16):
                    w = w16[j]
                    e = gg * 16 + j
                    for k in range(8):
                        rows_[b][e, pl.ds(k * 16, 16)] = (
                            rows_[b][e, pl.ds(k * 16, 16)] * w)
                return 0
            lax.fori_loop(0, 4, sb, 0)
            pltpu.async_copy(rows_[b], acc_sh.at[didx_[b]], sems_[b],
                             add=True)
        return 0
    lax.fori_loop(0, nch // 2, grp, 0)
    pltpu.make_async_copy(rows1, acc_sh.at[didx1], sems1).wait()
    plsc.subcore_barrier()
    pltpu.sync_copy(acc_sh.at[pl.ds(s * 640, 640)],
                    part_h.at[pl.ds(c * _NPE + s * 640, 640)])


@functools.partial(
    pl.kernel,
    out_type=jax.ShapeDtypeStruct((2 * _NPU, _DIM), jnp.float32),
    mesh=_mesh,
    compiler_params=_sc_params,
    scratch_types=[
        pltpu.VMEM_SHARED((_NPU, _DIM), jnp.float32),
        pltpu.VMEM((_UI_SUB * 128,), jnp.int32),
        pltpu.VMEM((64, _DIM), jnp.float32),
        pltpu.VMEM((64, _DIM), jnp.float32),
        pltpu.VMEM((64,), jnp.int32),
        pltpu.VMEM((64,), jnp.int32),
        pltpu.VMEM((64,), jnp.int32),
        pltpu.VMEM((64,), jnp.int32),
        pltpu.SemaphoreType.DMA,
        pltpu.SemaphoreType.DMA,
        pltpu.SemaphoreType.DMA,
        pltpu.SemaphoreType.DMA,
    ],
)
def _ui(pck_h, tab_h, usum_h,
        acc_sh, pck, rows0, rows1, sidx0, sidx1, didx0, didx1,
        semg0, semg1, sems0, sems1):
    c = lax.axis_index("c")
    s = lax.axis_index("s")
    n = _UI_SUB * 128
    nch = n // 64
    rows_ = (rows0, rows1)
    sidx_ = (sidx0, sidx1)
    didx_ = (didx0, didx1)
    semg_ = (semg0, semg1)
    sems_ = (sems0, sems1)
    pltpu.sync_copy(pck_h.at[pl.ds(s * n, n)], pck)
    z = jnp.zeros((16,), jnp.float32)
    def zb(i, _):
        for k in range(8):
            rows0[i, pl.ds(k * 16, 16)] = z
        return 0
    lax.fori_loop(0, 64, zb, 0)
    for i in range(8):
        pltpu.sync_copy(rows0, acc_sh.at[pl.ds(s * 512 + i * 64, 64)])
    plsc.subcore_barrier()

    off = c * (_NPE + _N_USERS)
    for i in range(4):
        p16 = pck[pl.ds(i * 16, 16)]
        sidx0[pl.ds(i * 16, 16)] = (p16 & 8191) + off
        didx0[pl.ds(i * 16, 16)] = p16 >> 13
    pltpu.async_copy(tab_h.at[sidx0], rows0, semg0)

    def grp(g, _):
        for b in range(2):
            b1 = 1 - b
            cc = g * 2 + b
            @pl.when(cc > 0)
            def _():
                pltpu.make_async_copy(rows_[b1], acc_sh.at[didx_[b1]],
                                      sems_[b1]).wait()
            @pl.when(cc + 1 < nch)
            def _():
                def ub(i, _):
                    p16 = pck[pl.ds((cc + 1) * 64 + i * 16, 16)]
                    sidx_[b1][pl.ds(i * 16, 16)] = (p16 & 8191) + off
                    didx_[b1][pl.ds(i * 16, 16)] = p16 >> 13
                    return 0
                lax.fori_loop(0, 4, ub, 0)
                pltpu.async_copy(tab_h.at[sidx_[b1]], rows_[b1], semg_[b1])
            pltpu.make_async_copy(tab_h.at[sidx_[b]], rows_[b],
                                  semg_[b]).wait()
            pltpu.async_copy(rows_[b], acc_sh.at[didx_[b]], sems_[b],
                             add=True)
        return 0
    lax.fori_loop(0, nch // 2, grp, 0)
    pltpu.make_async_copy(rows1, acc_sh.at[didx1], sems1).wait()
    plsc.subcore_barrier()
    pltpu.sync_copy(acc_sh.at[pl.ds(s * 512, 512)],
                    usum_h.at[pl.ds(c * _NPU + s * 512, 512)])


def _recip_body(p_ref, o_ref):
    cnt = p_ref[0] + p_ref[1]
    o_ref[...] = 1.0 / jnp.maximum(cnt, 1.0)


def _recip(parts):
    p = parts.reshape(2, _NBINS // 128, 128)
    out = pl.pallas_call(
        _recip_body,
        out_shape=jax.ShapeDtypeStruct((_NBINS // 128, 128), jnp.float32),
    )(p)
    return out.reshape(_NBINS)


def _comb_body(p0_ref, p1_ref, r_ref, t_ref, ro_ref):
    t = p0_ref[...] + p1_ref[...]
    t_ref[...] = t
    ro_ref[...] = r_ref[...] + t


def _combine(part, res_in):
    blk = 2048
    grid = _NPE // blk
    spec = pl.BlockSpec((blk, _DIM), lambda i: (i, 0))
    return pl.pallas_call(
        _comb_body,
        grid=(grid,),
        in_specs=[spec, spec, spec],
        out_specs=[spec, spec],
        out_shape=[
            jax.ShapeDtypeStruct((_NPE, _DIM), jnp.float32),
            jax.ShapeDtypeStruct((_NPE, _DIM), jnp.float32),
        ],
    )(part[:_NPE], part[_NPE:], res_in)


def _mean_body(r_ref, o_ref):
    rid = lax.broadcasted_iota(jnp.int32, (_NPE, _DIM), 0)
    m = jnp.sum(jnp.where(rid < _N_ITEMS, r_ref[...], 0.0), axis=0,
                keepdims=True) * (1.0 / _N_ITEMS)
    o_ref[...] = jnp.broadcast_to(m, (8, _DIM))


def _item_mean(res):
    return pl.pallas_call(
        _mean_body,
        out_shape=jax.ShapeDtypeStruct((8, _DIM), jnp.float32),
    )(res)


def _users_body(sf_ref, sf0_ref, rc_ref, ucf_ref, mi_ref, o_ref):
    rc = rc_ref[...]
    uf = sf_ref[...] * rc
    uf0 = sf0_ref[...] * rc
    ucf = ucf_ref[...]
    mi = mi_ref[0:1, :]
    cols = []
    for i in range(_N_REL):
        a, b = 32 * i, 32 * (i + 1)
        ua = jnp.sum(uf[:, a:b] * ucf[:, a:b], axis=1, keepdims=True)
        ua = jnp.maximum(ua, 0.0) + 1e-10
        ma = jnp.sum(mi[:, a:b] * ucf[:, a:b], axis=1, keepdims=True)
        ma = jnp.maximum(ma, 0.0) + 1e-08
        att = _K_ATT * jnp.maximum(ua / ma - 1.0, 0.0) + 0.01
        sc = jnp.tanh(att)
        cols.append(jnp.broadcast_to(sc, (sc.shape[0], 32)))
    score = jnp.concatenate(cols, axis=1)
    o_ref[...] = score * uf + uf0 + ucf


def _users(sf, sf0, rc, ucf, mi):
    blk = 4000
    grid = _N_USERS // blk
    spec = pl.BlockSpec((blk, _DIM), lambda i: (i, 0))
    spec1 = pl.BlockSpec((blk, 1), lambda i: (i, 0))
    specm = pl.BlockSpec((8, _DIM), lambda i: (0, 0))
    return pl.pallas_call(
        _users_body,
        grid=(grid,),
        in_specs=[spec, spec, spec1, spec, specm],
        out_specs=spec,
        out_shape=jax.ShapeDtypeStruct((_N_USERS, _DIM), jnp.float32),
    )(sf, sf0, rc, ucf, mi)


def _ent_body(r_ref, c_ref, o_ref):
    o_ref[...] = r_ref[...] + c_ref[...]


def _entity(res8k, cfe):
    blk = 2000
    grid = _N_ITEMS // blk
    spec = pl.BlockSpec((blk, _DIM), lambda i: (i, 0))
    return pl.pallas_call(
        _ent_body,
        grid=(grid,),
        in_specs=[spec, spec],
        out_specs=spec,
        out_shape=jax.ShapeDtypeStruct((_N_ITEMS, _DIM), jnp.float32),
    )(res8k, cfe)


def kernel(kg_edge_index, kg_edge_type, ui_edge_index, all_embed,
           all_embed_cf, dropout):
    src = kg_edge_index[0]
    dst = kg_edge_index[1]
    typ = kg_edge_type
    pad_kg = _EPKG - _E_KG
    src_p = jnp.concatenate([src, jnp.zeros((pad_kg,), jnp.int32)])
    dst_p = jnp.concatenate([dst, jnp.full((pad_kg,), _SENT_E, jnp.int32)])
    typ_p = jnp.concatenate([typ, jnp.zeros((pad_kg,), jnp.int32)])
    td_p = typ_p * _NPE + dst_p

    isrc = ui_edge_index[0]
    udst = ui_edge_index[1]
    pad_ui = _EPUI - _E_UI
    isrc_p = jnp.concatenate([isrc, jnp.zeros((pad_ui,), jnp.int32)])
    udst_p = jnp.concatenate([udst, jnp.full((pad_ui,), _SENT_U, jnp.int32)])

    pad_h = _EHIST - (_EPKG + _EPUI)
    hidx = jnp.concatenate(
        [td_p, udst_p + _N_REL * _NPE,
         jnp.full((pad_h,), _SENT_B, jnp.int32)])
    pck_kg = src_p + (dst_p << 14) + (typ_p << 28)
    pck_ui = isrc_p + (udst_p << 13)

    cnt_parts = _hist(hidx)
    recip = _recip(cnt_parts.reshape(2, _NBINS))
    recip_kg = recip[: _N_REL * _NPE]

    whop = _wts(td_p, recip_kg)
    part0 = _passb(pck_kg, whop, all_embed)
    e0, res = _combine(part0, jnp.zeros((_NPE, _DIM), jnp.float32))
    part1 = _hop(pck_kg, whop, e0)
    c1, res = _combine(part1, res)
    part2 = _hop(pck_kg, whop, c1)
    _, res = _combine(part2, res)

    usum = _ui(pck_ui, jnp.concatenate([res, all_embed_cf], axis=0))

    sf = jnp.pad(usum[:_N_ITEMS], ((0, _N_USERS - _N_ITEMS), (0, 0)))
    sf0 = jnp.pad(usum[_NPU:_NPU + _N_ITEMS],
                  ((0, _N_USERS - _N_ITEMS), (0, 0)))
    rcu = jnp.pad(recip[_N_REL * _NPE:_N_REL * _NPE + _N_ITEMS],
                  (0, _N_USERS - _N_ITEMS)).reshape(_N_USERS, 1)

    mi = _item_mean(res)
    ucf = all_embed_cf[:_N_USERS]
    user_out = _users(sf, sf0, rcu, ucf, mi)
    ent_out = _entity(res[:_N_ITEMS], all_embed_cf[_N_USERS:_N_USERS + _N_ITEMS])
    return jnp.concatenate([user_out, ent_out], axis=0)


# speedup vs baseline: 6.6825x; 1.0157x over previous
import functools

import jax
import jax.numpy as jnp
from jax import lax
from jax.experimental import pallas as pl
from jax.experimental.pallas import tpu as pltpu
from jax.experimental.pallas import tpu_sc as plsc

_DIM = 128
_N_USERS = 20000
_N_ITEMS = 8000
_N_ENT = 10000
_N_REL = 4
_K_ATT = 0.5
_E_KG = 320000
_E_UI = 500000

_NPE = 10240
_SENT_E = 10200
_NPU = 8192
_SENT_U = 8100
_NBINS = _N_REL * _NPE + _NPU
_SENT_B = _N_REL * _NPE + _SENT_U

_KG_SUB = 80
_EPKG = 32 * _KG_SUB * 128
_UI_SUB = 248
_EPUI = 16 * _UI_SUB * 128
_HROWS = 208
_EHIST = 32 * _HROWS * 128

_mesh = plsc.VectorSubcoreMesh(core_axis_name="c", subcore_axis_name="s")
_sc_params = pltpu.CompilerParams(needs_layout_passes=False)


@functools.partial(
    pl.kernel,
    out_type=jax.ShapeDtypeStruct((2 * _NBINS,), jnp.float32),
    mesh=_mesh,
    compiler_params=_sc_params,
    scratch_types=[
        pltpu.VMEM_SHARED((_NBINS,), jnp.float32),
        pltpu.VMEM((128,), jnp.int32),
        pltpu.VMEM((128,), jnp.float32),
        pltpu.VMEM((_NBINS // 16,), jnp.float32),
    ],
)
def _hist(hidx_h, out_h, cnt_sh, hidx_v, ones_v, zv):
    c = lax.axis_index("c")
    s = lax.axis_index("s")
    wid = s * 2 + c
    one = jnp.ones((16,), jnp.float32)
    for i in range(8):
        ones_v[pl.ds(i * 16, 16)] = one
    z = jnp.zeros((16,), jnp.float32)
    def zb(i, _):
        zv[pl.ds(i * 16, 16)] = z
        return 0
    lax.fori_loop(0, _NBINS // 16 // 16, zb, 0)
    pltpu.sync_copy(zv, cnt_sh.at[pl.ds(s * (_NBINS // 16), _NBINS // 16)])
    plsc.subcore_barrier()
    base = wid * _HROWS * 128
    def hb(r, _):
        pltpu.sync_copy(hidx_h.at[pl.ds(base + r * 128, 128)], hidx_v)
        pltpu.sync_copy(ones_v, cnt_sh.at[hidx_v], add=True)
        return 0
    lax.fori_loop(0, _HROWS, hb, 0)
    plsc.subcore_barrier()
    sl = _NBINS // 16
    pltpu.sync_copy(cnt_sh.at[pl.ds(s * sl, sl)],
                    out_h.at[pl.ds(c * _NBINS + s * sl, sl)])


@functools.partial(
    pl.kernel,
    out_type=jax.ShapeDtypeStruct((_EPKG,), jnp.float32),
    mesh=_mesh,
    compiler_params=_sc_params,
    scratch_types=[
        pltpu.VMEM((_N_REL * _NPE,), jnp.float32),
        pltpu.VMEM((_KG_SUB * 128,), jnp.int32),
        pltpu.VMEM((_KG_SUB * 128,), jnp.float32),
    ],
)
def _wts(td_h, recip_h, whop_h, rtab, tdv, wv):
    c = lax.axis_index("c")
    s = lax.axis_index("s")
    wid = s * 2 + c
    n = _KG_SUB * 128
    pltpu.sync_copy(recip_h, rtab)
    pltpu.sync_copy(td_h.at[pl.ds(wid * n, n)], tdv)
    def wb(i, _):
        idx16 = tdv[pl.ds(i * 16, 16)]
        wv[pl.ds(i * 16, 16)] = plsc.load_gather(rtab, [idx16])
        return 0
    lax.fori_loop(0, n // 16, wb, 0)
    pltpu.sync_copy(wv, whop_h.at[pl.ds(wid * n, n)])


@functools.partial(
    pl.kernel,
    out_type=jax.ShapeDtypeStruct((2 * _NPE, _DIM), jnp.float32),
    mesh=_mesh,
    compiler_params=_sc_params,
    scratch_types=[
        pltpu.VMEM_SHARED((_NPE, _DIM), jnp.float32),
        pltpu.VMEM((_KG_SUB * 128,), jnp.int32),
        pltpu.VMEM((_KG_SUB * 128,), jnp.float32),
        pltpu.VMEM((64, _DIM), jnp.float32),
        pltpu.VMEM((64, _DIM), jnp.float32),
        pltpu.VMEM((64,), jnp.int32),
        pltpu.VMEM((64,), jnp.int32),
        pltpu.VMEM((64,), jnp.int32),
        pltpu.VMEM((64,), jnp.int32),
        pltpu.SemaphoreType.DMA,
        pltpu.SemaphoreType.DMA,
        pltpu.SemaphoreType.DMA,
        pltpu.SemaphoreType.DMA,
    ],
)
def _passb(pck_h, whop_h, emb_h, part_h,
           acc_sh, pck, wvb, rows0, rows1, sidx0, sidx1, didx0, didx1,
           semg0, semg1, sems0, sems1):
    c = lax.axis_index("c")
    s = lax.axis_index("s")
    wid = s * 2 + c
    n = _KG_SUB * 128
    nch = n // 64
    rows_ = (rows0, rows1)
    sidx_ = (sidx0, sidx1)
    didx_ = (didx0, didx1)
    semg_ = (semg0, semg1)
    sems_ = (sems0, sems1)
    pltpu.sync_copy(pck_h.at[pl.ds(wid * n, n)], pck)
    pltpu.sync_copy(whop_h.at[pl.ds(wid * n, n)], wvb)
    z = jnp.zeros((16,), jnp.float32)
    def zb(i, _):
        for k in range(8):
            rows0[i, pl.ds(k * 16, 16)] = z
        return 0
    lax.fori_loop(0, 64, zb, 0)
    for i in range(10):
        pltpu.sync_copy(rows0, acc_sh.at[pl.ds(s * 640 + i * 64, 64)])
    plsc.subcore_barrier()

    for i in range(4):
        p16 = pck[pl.ds(i * 16, 16)]
        sidx0[pl.ds(i * 16, 16)] = p16 & 16383
        didx0[pl.ds(i * 16, 16)] = (p16 >> 14) & 16383
    pltpu.async_copy(emb_h.at[sidx0], rows0, semg0)

    def grp(g, _):
        for b in range(2):
            b1 = 1 - b
            cc = g * 2 + b
            @pl.when(cc > 0)
            def _():
                pltpu.make_async_copy(rows_[b1], acc_sh.at[didx_[b1]],
                                      sems_[b1]).wait()
            @pl.when(cc + 1 < nch)
            def _():
                def ub(i, _):
                    p16 = pck[pl.ds((cc + 1) * 64 + i * 16, 16)]
                    sidx_[b1][pl.ds(i * 16, 16)] = p16 & 16383
                    didx_[b1][pl.ds(i * 16, 16)] = (p16 >> 14) & 16383
                    return 0
                lax.fori_loop(0, 4, ub, 0)
                pltpu.async_copy(emb_h.at[sidx_[b1]], rows_[b1], semg_[b1])
            pltpu.make_async_copy(emb_h.at[sidx_[b]], rows_[b],
                                  semg_[b]).wait()
            def sb(gg, _):
                w16 = wvb[pl.ds(cc * 64 + gg * 16, 16)]
                t16 = pck[pl.ds(cc * 64 + gg * 16, 16)] >> 28
                for j in range(16):
                    w = w16[j]
                    t = t16[j]
                    e = gg * 16 + j
                    for k in range(8):
                        wk = jnp.where(t == (k // 2), w, jnp.float32(0.0))
                        rows_[b][e, pl.ds(k * 16, 16)] = (
                            rows_[b][e, pl.ds(k * 16, 16)] * wk)
                return 0
            lax.fori_loop(0, 4, sb, 0)
            pltpu.async_copy(rows_[b], acc_sh.at[didx_[b]], sems_[b],
                             add=True)
        return 0
    lax.fori_loop(0, nch // 2, grp, 0)
    pltpu.make_async_copy(rows1, acc_sh.at[didx1], sems1).wait()
    plsc.subcore_barrier()
    pltpu.sync_copy(acc_sh.at[pl.ds(s * 640, 640)],
                    part_h.at[pl.ds(c * _NPE + s * 640, 640)])


@functools.partial(
    pl.kernel,
    out_type=jax.ShapeDtypeStruct((2 * _NPE, _DIM), jnp.float32),
    mesh=_mesh,
    compiler_params=_sc_params,
    scratch_types=[
        pltpu.VMEM_SHARED((_NPE, _DIM), jnp.float32),
        pltpu.VMEM((_KG_SUB * 128,), jnp.int32),
        pltpu.VMEM((_KG_SUB * 128,), jnp.float32),
        pltpu.VMEM((64, _DIM), jnp.float32),
        pltpu.VMEM((64, _DIM), jnp.float32),
        pltpu.VMEM((64,), jnp.int32),
        pltpu.VMEM((64,), jnp.int32),
        pltpu.VMEM((64,), jnp.int32),
        pltpu.VMEM((64,), jnp.int32),
        pltpu.SemaphoreType.DMA,
        pltpu.SemaphoreType.DMA,
        pltpu.SemaphoreType.DMA,
        pltpu.SemaphoreType.DMA,
    ],
)
def _hop(pck_h, whop_h, tab_h, part_h,
         acc_sh, pck, wvb, rows0, rows1, sidx0, sidx1, didx0, didx1,
         semg0, semg1, sems0, sems1):
    c = lax.axis_index("c")
    s = lax.axis_index("s")
    wid = s * 2 + c
    n = _KG_SUB * 128
    nch = n // 64
    rows_ = (rows0, rows1)
    sidx_ = (sidx0, sidx1)
    didx_ = (didx0, didx1)
    semg_ = (semg0, semg1)
    sems_ = (sems0, sems1)
    pltpu.sync_copy(pck_h.at[pl.ds(wid * n, n)], pck)
    pltpu.sync_copy(whop_h.at[pl.ds(wid * n, n)], wvb)
    z = jnp.zeros((16,), jnp.float32)
    def zb(i, _):
        for k in range(8):
            rows0[i, pl.ds(k * 16, 16)] = z
        return 0
    lax.fori_loop(0, 64, zb, 0)
    for i in range(10):
        pltpu.sync_copy(rows0, acc_sh.at[pl.ds(s * 640 + i * 64, 64)])
    plsc.subcore_barrier()

    for i in range(4):
        p16 = pck[pl.ds(i * 16, 16)]
        sidx0[pl.ds(i * 16, 16)] = p16 & 16383
        didx0[pl.ds(i * 16, 16)] = (p16 >> 14) & 16383
    pltpu.async_copy(tab_h.at[sidx0], rows0, semg0)

    def grp(g, _):
        for b in range(2):
            b1 = 1 - b
            cc = g * 2 + b
            @pl.when(cc > 0)
            def _():
                pltpu.make_async_copy(rows_[b1], acc_sh.at[didx_[b1]],
                                      sems_[b1]).wait()
            @pl.when(cc + 1 < nch)
            def _():
                def ub(i, _):
                    p16 = pck[pl.ds((cc + 1) * 64 + i * 16, 16)]
                    sidx_[b1][pl.ds(i * 16, 16)] = p16 & 16383
                    didx_[b1][pl.ds(i * 16, 16)] = (p16 >> 14) & 16383
                    return 0
                lax.fori_loop(0, 4, ub, 0)
                pltpu.async_copy(tab_h.at[sidx_[b1]], rows_[b1], semg_[b1])
            pltpu.make_async_copy(tab_h.at[sidx_[b]], rows_[b],
                                  semg_[b]).wait()
            def sb(gg, _):
                w16 = wvb[pl.ds(cc * 64 + gg * 16, 16)]
                for j in range(16):
                    w = w16[j]
                    e = gg * 16 + j
                    for k in range(8):
                        rows_[b][e, pl.ds(k * 16, 16)] = (
                            rows_[b][e, pl.ds(k * 16, 16)] * w)
                return 0
            lax.fori_loop(0, 4, sb, 0)
            pltpu.async_copy(rows_[b], acc_sh.at[didx_[b]], sems_[b],
                             add=True)
        return 0
    lax.fori_loop(0, nch // 2, grp, 0)
    pltpu.make_async_copy(rows1, acc_sh.at[didx1], sems1).wait()
    plsc.subcore_barrier()
    pltpu.sync_copy(acc_sh.at[pl.ds(s * 640, 640)],
                    part_h.at[pl.ds(c * _NPE + s * 640, 640)])


@functools.partial(
    pl.kernel,
    out_type=jax.ShapeDtypeStruct((2 * _NPU, _DIM), jnp.float32),
    mesh=_mesh,
    compiler_params=_sc_params,
    scratch_types=[
        pltpu.VMEM_SHARED((_NPU, _DIM), jnp.float32),
        pltpu.VMEM((_UI_SUB * 128,), jnp.int32),
        pltpu.VMEM((128, _DIM), jnp.float32),
        pltpu.VMEM((128, _DIM), jnp.float32),
        pltpu.VMEM((128,), jnp.int32),
        pltpu.VMEM((128,), jnp.int32),
        pltpu.VMEM((128,), jnp.int32),
        pltpu.VMEM((128,), jnp.int32),
        pltpu.SemaphoreType.DMA,
        pltpu.SemaphoreType.DMA,
        pltpu.SemaphoreType.DMA,
        pltpu.SemaphoreType.DMA,
    ],
)
def _ui(pck_h, tab_h, usum_h,
        acc_sh, pck, rows0, rows1, sidx0, sidx1, didx0, didx1,
        semg0, semg1, sems0, sems1):
    c = lax.axis_index("c")
    s = lax.axis_index("s")
    n = _UI_SUB * 128
    nch = n // 128
    rows_ = (rows0, rows1)
    sidx_ = (sidx0, sidx1)
    didx_ = (didx0, didx1)
    semg_ = (semg0, semg1)
    sems_ = (sems0, sems1)
    pltpu.sync_copy(pck_h.at[pl.ds(s * n, n)], pck)
    z = jnp.zeros((16,), jnp.float32)
    def zb(i, _):
        for k in range(8):
            rows0[i, pl.ds(k * 16, 16)] = z
        return 0
    lax.fori_loop(0, 128, zb, 0)
    for i in range(4):
        pltpu.sync_copy(rows0, acc_sh.at[pl.ds(s * 512 + i * 128, 128)])
    plsc.subcore_barrier()

    off = c * (_NPE + _N_USERS)
    for i in range(8):
        p16 = pck[pl.ds(i * 16, 16)]
        sidx0[pl.ds(i * 16, 16)] = (p16 & 8191) + off
        didx0[pl.ds(i * 16, 16)] = p16 >> 13
    pltpu.async_copy(tab_h.at[sidx0], rows0, semg0)

    def grp(g, _):
        for b in range(2):
            b1 = 1 - b
            cc = g * 2 + b
            @pl.when(cc > 0)
            def _():
                pltpu.make_async_copy(rows_[b1], acc_sh.at[didx_[b1]],
                                      sems_[b1]).wait()
            @pl.when(cc + 1 < nch)
            def _():
                def ub(i, _):
                    p16 = pck[pl.ds((cc + 1) * 128 + i * 16, 16)]
                    sidx_[b1][pl.ds(i * 16, 16)] = (p16 & 8191) + off
                    didx_[b1][pl.ds(i * 16, 16)] = p16 >> 13
                    return 0
                lax.fori_loop(0, 8, ub, 0)
                pltpu.async_copy(tab_h.at[sidx_[b1]], rows_[b1], semg_[b1])
            pltpu.make_async_copy(tab_h.at[sidx_[b]], rows_[b],
                                  semg_[b]).wait()
            pltpu.async_copy(rows_[b], acc_sh.at[didx_[b]], sems_[b],
                             add=True)
        return 0
    lax.fori_loop(0, nch // 2, grp, 0)
    pltpu.make_async_copy(rows1, acc_sh.at[didx1], sems1).wait()
    plsc.subcore_barrier()
    pltpu.sync_copy(acc_sh.at[pl.ds(s * 512, 512)],
                    usum_h.at[pl.ds(c * _NPU + s * 512, 512)])


def _recip_body(p_ref, o_ref):
    cnt = p_ref[0] + p_ref[1]
    o_ref[...] = 1.0 / jnp.maximum(cnt, 1.0)


def _recip(parts):
    p = parts.reshape(2, _NBINS // 128, 128)
    out = pl.pallas_call(
        _recip_body,
        out_shape=jax.ShapeDtypeStruct((_NBINS // 128, 128), jnp.float32),
    )(p)
    return out.reshape(_NBINS)


def _comb_body(p0_ref, p1_ref, r_ref, t_ref, ro_ref):
    t = p0_ref[...] + p1_ref[...]
    t_ref[...] = t
    ro_ref[...] = r_ref[...] + t


def _combine(part, res_in):
    blk = 2048
    grid = _NPE // blk
    spec = pl.BlockSpec((blk, _DIM), lambda i: (i, 0))
    return pl.pallas_call(
        _comb_body,
        grid=(grid,),
        in_specs=[spec, spec, spec],
        out_specs=[spec, spec],
        out_shape=[
            jax.ShapeDtypeStruct((_NPE, _DIM), jnp.float32),
            jax.ShapeDtypeStruct((_NPE, _DIM), jnp.float32),
        ],
    )(part[:_NPE], part[_NPE:], res_in)


def _mean_body(r_ref, o_ref):
    rid = lax.broadcasted_iota(jnp.int32, (_NPE, _DIM), 0)
    m = jnp.sum(jnp.where(rid < _N_ITEMS, r_ref[...], 0.0), axis=0,
                keepdims=True) * (1.0 / _N_ITEMS)
    o_ref[...] = jnp.broadcast_to(m, (8, _DIM))


def _item_mean(res):
    return pl.pallas_call(
        _mean_body,
        out_shape=jax.ShapeDtypeStruct((8, _DIM), jnp.float32),
    )(res)


def _users_body(sf_ref, sf0_ref, rc_ref, ucf_ref, mi_ref, o_ref):
    rc = rc_ref[...]
    uf = sf_ref[...] * rc
    uf0 = sf0_ref[...] * rc
    ucf = ucf_ref[...]
    mi = mi_ref[0:1, :]
    cols = []
    for i in range(_N_REL):
        a, b = 32 * i, 32 * (i + 1)
        ua = jnp.sum(uf[:, a:b] * ucf[:, a:b], axis=1, keepdims=True)
        ua = jnp.maximum(ua, 0.0) + 1e-10
        ma = jnp.sum(mi[:, a:b] * ucf[:, a:b], axis=1, keepdims=True)
        ma = jnp.maximum(ma, 0.0) + 1e-08
        att = _K_ATT * jnp.maximum(ua / ma - 1.0, 0.0) + 0.01
        sc = jnp.tanh(att)
        cols.append(jnp.broadcast_to(sc, (sc.shape[0], 32)))
    score = jnp.concatenate(cols, axis=1)
    o_ref[...] = score * uf + uf0 + ucf


def _users(sf, sf0, rc, ucf, mi):
    blk = 4000
    grid = _N_USERS // blk
    spec = pl.BlockSpec((blk, _DIM), lambda i: (i, 0))
    spec1 = pl.BlockSpec((blk, 1), lambda i: (i, 0))
    specm = pl.BlockSpec((8, _DIM), lambda i: (0, 0))
    return pl.pallas_call(
        _users_body,
        grid=(grid,),
        in_specs=[spec, spec, spec1, spec, specm],
        out_specs=spec,
        out_shape=jax.ShapeDtypeStruct((_N_USERS, _DIM), jnp.float32),
    )(sf, sf0, rc, ucf, mi)


def _ent_body(r_ref, c_ref, o_ref):
    o_ref[...] = r_ref[...] + c_ref[...]


def _entity(res8k, cfe):
    blk = 2000
    grid = _N_ITEMS // blk
    spec = pl.BlockSpec((blk, _DIM), lambda i: (i, 0))
    return pl.pallas_call(
        _ent_body,
        grid=(grid,),
        in_specs=[spec, spec],
        out_specs=spec,
        out_shape=jax.ShapeDtypeStruct((_N_ITEMS, _DIM), jnp.float32),
    )(res8k, cfe)


def kernel(kg_edge_index, kg_edge_type, ui_edge_index, all_embed,
           all_embed_cf, dropout):
    src = kg_edge_index[0]
    dst = kg_edge_index[1]
    typ = kg_edge_type
    pad_kg = _EPKG - _E_KG
    src_p = jnp.concatenate([src, jnp.zeros((pad_kg,), jnp.int32)])
    dst_p = jnp.concatenate([dst, jnp.full((pad_kg,), _SENT_E, jnp.int32)])
    typ_p = jnp.concatenate([typ, jnp.zeros((pad_kg,), jnp.int32)])
    td_p = typ_p * _NPE + dst_p

    isrc = ui_edge_index[0]
    udst = ui_edge_index[1]
    pad_ui = _EPUI - _E_UI
    isrc_p = jnp.concatenate([isrc, jnp.zeros((pad_ui,), jnp.int32)])
    udst_p = jnp.concatenate([udst, jnp.full((pad_ui,), _SENT_U, jnp.int32)])

    pad_h = _EHIST - (_EPKG + _EPUI)
    hidx = jnp.concatenate(
        [td_p, udst_p + _N_REL * _NPE,
         jnp.full((pad_h,), _SENT_B, jnp.int32)])
    pck_kg = src_p + (dst_p << 14) + (typ_p << 28)
    pck_ui = isrc_p + (udst_p << 13)

    cnt_parts = _hist(hidx)
    recip = _recip(cnt_parts.reshape(2, _NBINS))
    recip_kg = recip[: _N_REL * _NPE]

    whop = _wts(td_p, recip_kg)
    part0 = _passb(pck_kg, whop, all_embed)
    e0, res = _combine(part0, jnp.zeros((_NPE, _DIM), jnp.float32))
    part1 = _hop(pck_kg, whop, e0)
    c1, res = _combine(part1, res)
    part2 = _hop(pck_kg, whop, c1)
    _, res = _combine(part2, res)

    usum = _ui(pck_ui, jnp.concatenate([res, all_embed_cf], axis=0))

    sf = jnp.pad(usum[:_N_ITEMS], ((0, _N_USERS - _N_ITEMS), (0, 0)))
    sf0 = jnp.pad(usum[_NPU:_NPU + _N_ITEMS],
                  ((0, _N_USERS - _N_ITEMS), (0, 0)))
    rcu = jnp.pad(recip[_N_REL * _NPE:_N_REL * _NPE + _N_ITEMS],
                  (0, _N_USERS - _N_ITEMS)).reshape(_N_USERS, 1)

    mi = _item_mean(res)
    ucf = all_embed_cf[:_N_USERS]
    user_out = _users(sf, sf0, rcu, ucf, mi)
    ent_out = _entity(res[:_N_ITEMS], all_embed_cf[_N_USERS:_N_USERS + _N_ITEMS])
    return jnp.concatenate([user_out, ent_out], axis=0)
